# baseline (speedup 1.0000x reference)
import functools

import jax
import jax.numpy as jnp
from jax import lax
from jax.experimental import pallas as pl
from jax.experimental.pallas import tpu as pltpu
from jax.experimental.pallas import tpu_sc as plsc

NQ = 4096
NK = 65536
QT = 512
KT = 2048
PAD_D = 128

SC_CORES = 2
SC_SUBCORES = 16
SC_WORKERS = SC_CORES * SC_SUBCORES
ROWS_PER_WORKER = NQ // SC_WORKERS


def _nn_body(pm2_ref, gtt_ref, idx_ref, run_min, run_idx):
    k = pl.program_id(1)

    @pl.when(k == 0)
    def _init():
        run_min[...] = jnp.full((QT, 1), jnp.inf, jnp.float32)
        run_idx[...] = jnp.zeros((QT, 1), jnp.int32)

    g = gtt_ref[...]
    g2 = jnp.sum(g * g, axis=0, keepdims=True)
    s = lax.dot_general(
        pm2_ref[...], g,
        dimension_numbers=(((1,), (0,)), ((), ())),
        preferred_element_type=jnp.float32,
        precision=lax.Precision.HIGHEST,
    ) + g2

    m = jnp.min(s, axis=1, keepdims=True)
    lane = lax.broadcasted_iota(jnp.int32, s.shape, 1)
    cand = jnp.where(s == m, lane, jnp.int32(NK))
    tile_idx = jnp.min(cand, axis=1, keepdims=True) + k * KT

    better = m < run_min[...]
    run_idx[...] = jnp.where(better, tile_idx, run_idx[...])
    run_min[...] = jnp.where(better, m, run_min[...])

    @pl.when(k == pl.num_programs(1) - 1)
    def _emit():
        idx_ref[...] = run_idx[...]


_nn_call = pl.pallas_call(
    _nn_body,
    grid=(NQ // QT, NK // KT),
    in_specs=[
        pl.BlockSpec((QT, 3), lambda q, k: (q, 0)),
        pl.BlockSpec((3, KT), lambda q, k: (0, k)),
    ],
    out_specs=pl.BlockSpec((QT, 1), lambda q, k: (q, 0)),
    out_shape=jax.ShapeDtypeStruct((NQ, 1), jnp.int32),
    scratch_shapes=[
        pltpu.VMEM((QT, 1), jnp.float32),
        pltpu.VMEM((QT, 1), jnp.int32),
    ],
)


def _sc_gather_body(table_hbm, idx_hbm, out_hbm, idx_v, rows_v, sem):
    wid = lax.axis_index("s") * SC_CORES + lax.axis_index("c")
    base = wid * ROWS_PER_WORKER
    pltpu.sync_copy(idx_hbm.at[pl.ds(base, ROWS_PER_WORKER)], idx_v)
    pltpu.async_copy(table_hbm.at[idx_v], rows_v, sem).wait()
    pltpu.sync_copy(rows_v, out_hbm.at[pl.ds(base, ROWS_PER_WORKER)])


def _sc_gather(table, idx):
    kern = pl.kernel(
        _sc_gather_body,
        out_type=jax.ShapeDtypeStruct((NQ, PAD_D), jnp.float32),
        mesh=plsc.VectorSubcoreMesh(core_axis_name="c", subcore_axis_name="s"),
        scratch_types=[
            pltpu.VMEM((ROWS_PER_WORKER,), jnp.int32),
            pltpu.VMEM((ROWS_PER_WORKER, PAD_D), jnp.float32),
            pltpu.SemaphoreType.DMA,
        ],
    )
    return kern(table, idx)


def _loss_body(pft_ref, nrt_ref, out_ref):
    pf = pft_ref[...]
    nr = nrt_ref[...]
    p = pf[0:3, :]
    pn = pf[3:6, :]
    g = nr[0:3, :]
    gn = nr[3:6, :]

    d = p - g
    inlier = jnp.sum(d * d, keepdims=True) * (1.0 / (NQ * 3))

    pn_norm = jnp.maximum(jnp.sqrt(jnp.sum(pn * pn, axis=0, keepdims=True)), 1e-4)
    gn_norm = jnp.maximum(jnp.sqrt(jnp.sum(gn * gn, axis=0, keepdims=True)), 1e-4)
    cos = jnp.sum((pn / pn_norm) * (gn / gn_norm), axis=0, keepdims=True)
    norm_loss = jnp.sum(1.0 - cos, axis=1, keepdims=True) * (1.0 / NQ)

    out_ref[...] = inlier + norm_loss


_loss_call = pl.pallas_call(
    _loss_body,
    in_specs=[
        pl.BlockSpec((6, NQ), lambda: (0, 0)),
        pl.BlockSpec((PAD_D, NQ), lambda: (0, 0)),
    ],
    out_specs=pl.BlockSpec((1, 1), lambda: (0, 0)),
    out_shape=jax.ShapeDtypeStruct((1, 1), jnp.float32),
)


def kernel(pred_feat, pred_decoder, input_data, gt_data):
    del pred_decoder, input_data
    pred_points = pred_feat[:, :3]
    gauss = jax.random.normal(jax.random.key(1), pred_points.shape,
                              dtype=pred_points.dtype)
    pred_points = jnp.where(jnp.any(jnp.isnan(pred_points)), gauss, pred_points)

    pm2 = -2.0 * pred_points
    gtt = gt_data[:, :3].T
    idx = _nn_call(pm2, gtt)

    table = jnp.pad(gt_data, ((0, 0), (0, PAD_D - 6)))
    rows = _sc_gather(table, idx.reshape(NQ))

    pft = jnp.concatenate([pred_points, pred_feat[:, 3:]], axis=1).T
    loss = _loss_call(pft, rows.T)
    return loss[0, 0]


# speedup vs baseline: 1.1353x; 1.1353x over previous
import functools

import jax
import jax.numpy as jnp
from jax import lax
from jax.experimental import pallas as pl
from jax.experimental.pallas import tpu as pltpu
from jax.experimental.pallas import tpu_sc as plsc

NQ = 4096
NK = 65536
QT = 512
KT = 2048
PAD_D = 128

SC_CORES = 2
SC_SUBCORES = 16
SC_WORKERS = SC_CORES * SC_SUBCORES
ROWS_PER_WORKER = NQ // SC_WORKERS


LANE_BITS = 11
LANE_MASK = (1 << LANE_BITS) - 1
KEY_MASK = ~LANE_MASK


def _nn_body(paug_ref, gaug_ref, idx_ref, run_key, run_idx):
    k = pl.program_id(1)

    @pl.when(k == 0)
    def _init():
        run_key[...] = jnp.full((QT, 1), jnp.int32(0x7FFFFFFF), jnp.int32)
        run_idx[...] = jnp.zeros((QT, 1), jnp.int32)

    d2 = lax.dot_general(
        paug_ref[...], gaug_ref[...],
        dimension_numbers=(((1,), (0,)), ((), ())),
        preferred_element_type=jnp.float32,
        precision=lax.Precision.HIGHEST,
    )

    bits = lax.bitcast_convert_type(d2, jnp.int32)
    lane = lax.broadcasted_iota(jnp.int32, d2.shape, 1)
    key = (bits & jnp.int32(KEY_MASK)) | lane
    kmin = jnp.min(key, axis=1, keepdims=True)

    better = kmin < run_key[...]
    run_idx[...] = jnp.where(
        better, (kmin & jnp.int32(LANE_MASK)) + k * KT, run_idx[...])
    run_key[...] = jnp.where(better, kmin, run_key[...])

    @pl.when(k == pl.num_programs(1) - 1)
    def _emit():
        idx_ref[...] = run_idx[...]


_nn_call = pl.pallas_call(
    _nn_body,
    grid=(NQ // QT, NK // KT),
    in_specs=[
        pl.BlockSpec((QT, 8), lambda q, k: (q, 0)),
        pl.BlockSpec((8, KT), lambda q, k: (0, k)),
    ],
    out_specs=pl.BlockSpec((QT, 1), lambda q, k: (q, 0)),
    out_shape=jax.ShapeDtypeStruct((NQ, 1), jnp.int32),
    scratch_shapes=[
        pltpu.VMEM((QT, 1), jnp.int32),
        pltpu.VMEM((QT, 1), jnp.int32),
    ],
)


def _sc_gather_body(table_hbm, idx_hbm, out_hbm, idx_v, rows_v, sem):
    wid = lax.axis_index("s") * SC_CORES + lax.axis_index("c")
    base = wid * ROWS_PER_WORKER
    pltpu.sync_copy(idx_hbm.at[pl.ds(base, ROWS_PER_WORKER)], idx_v)
    pltpu.async_copy(table_hbm.at[idx_v], rows_v, sem).wait()
    pltpu.sync_copy(rows_v, out_hbm.at[pl.ds(base, ROWS_PER_WORKER)])


def _sc_gather(table, idx):
    kern = pl.kernel(
        _sc_gather_body,
        out_type=jax.ShapeDtypeStruct((NQ, PAD_D), jnp.float32),
        mesh=plsc.VectorSubcoreMesh(core_axis_name="c", subcore_axis_name="s"),
        scratch_types=[
            pltpu.VMEM((ROWS_PER_WORKER,), jnp.int32),
            pltpu.VMEM((ROWS_PER_WORKER, PAD_D), jnp.float32),
            pltpu.SemaphoreType.DMA,
        ],
    )
    return kern(table, idx)


def _loss_body(pft_ref, nrt_ref, out_ref):
    pf = pft_ref[...]
    nr = nrt_ref[...]
    p = pf[0:3, :]
    pn = pf[3:6, :]
    g = nr[0:3, :]
    gn = nr[3:6, :]

    d = p - g
    inlier = jnp.sum(d * d, keepdims=True) * (1.0 / (NQ * 3))

    pn_norm = jnp.maximum(jnp.sqrt(jnp.sum(pn * pn, axis=0, keepdims=True)), 1e-4)
    gn_norm = jnp.maximum(jnp.sqrt(jnp.sum(gn * gn, axis=0, keepdims=True)), 1e-4)
    cos = jnp.sum((pn / pn_norm) * (gn / gn_norm), axis=0, keepdims=True)
    norm_loss = jnp.sum(1.0 - cos, axis=1, keepdims=True) * (1.0 / NQ)

    out_ref[...] = inlier + norm_loss


_loss_call = pl.pallas_call(
    _loss_body,
    in_specs=[
        pl.BlockSpec((6, NQ), lambda: (0, 0)),
        pl.BlockSpec((PAD_D, NQ), lambda: (0, 0)),
    ],
    out_specs=pl.BlockSpec((1, 1), lambda: (0, 0)),
    out_shape=jax.ShapeDtypeStruct((1, 1), jnp.float32),
)


def kernel(pred_feat, pred_decoder, input_data, gt_data):
    del pred_decoder, input_data
    pred_points = pred_feat[:, :3]
    gauss = jax.random.normal(jax.random.key(1), pred_points.shape,
                              dtype=pred_points.dtype)
    pred_points = jnp.where(jnp.any(jnp.isnan(pred_points)), gauss, pred_points)

    p2 = jnp.sum(pred_points * pred_points, axis=1, keepdims=True)
    ones_q = jnp.ones((NQ, 1), jnp.float32)
    paug = jnp.concatenate(
        [pred_points, p2, ones_q, jnp.zeros((NQ, 3), jnp.float32)], axis=1)

    gt_pts = gt_data[:, :3]
    g2 = jnp.sum(gt_pts * gt_pts, axis=1, keepdims=True)
    ones_k = jnp.ones((NK, 1), jnp.float32)
    gaug = jnp.concatenate(
        [-2.0 * gt_pts, ones_k, g2, jnp.zeros((NK, 3), jnp.float32)], axis=1).T

    idx = _nn_call(paug, gaug)

    table = jnp.pad(gt_data, ((0, 0), (0, PAD_D - 6)))
    rows = _sc_gather(table, idx.reshape(NQ))

    pft = jnp.concatenate([pred_points, pred_feat[:, 3:]], axis=1).T
    loss = _loss_call(pft, rows.T)
    return loss[0, 0]
